# Initial kernel scaffold; baseline (speedup 1.0000x reference)
import jax
import jax.numpy as jnp
from jax.experimental import pallas as pl


def kernel(x, route_feature, task_id, params):
    raise NotImplementedError("write your pallas kernel here")


# speedup vs baseline: 1.4061x; 1.4061x over previous
import functools

import jax
import jax.numpy as jnp
from jax.experimental import pallas as pl
from jax.experimental.pallas import tpu as pltpu

_PREC = jax.lax.Precision.HIGHEST
_NE = 8


def _dot(a, b):
    return jnp.dot(a, b, precision=_PREC, preferred_element_type=jnp.float32)


def _conv3x3_kernel(x0_ref, x1_ref, x2_ref, w_ref, b_ref, out_ref, *, Hb, W):
    cout = w_ref.shape[-1]
    acc = jnp.zeros((Hb * W, cout), jnp.float32)
    for dy, xr in enumerate((x0_ref, x1_ref, x2_ref)):
        xb = xr[0]
        for dx in range(3):
            win = xb[:, dx:dx + W, :]
            acc = acc + _dot(win.reshape(Hb * W, win.shape[-1]), w_ref[dy, dx])
    out_ref[0] = acc + b_ref[...]


def _conv3x3(x_nhwc, w_oihw, bias):
    B, H, W, cin = x_nhwc.shape
    cout = w_oihw.shape[0]
    Hb = 14
    xp = jnp.pad(x_nhwc, ((0, 0), (1, 1), (1, 1), (0, 0)))
    xs = [xp[:, dy:dy + H, :, :] for dy in range(3)]
    w = w_oihw.transpose(2, 3, 1, 0)
    b = bias if bias is not None else jnp.zeros((cout,), jnp.float32)
    row_spec = pl.BlockSpec((1, Hb, W + 2, cin), lambda bi, hi: (bi, hi, 0, 0))
    return pl.pallas_call(
        functools.partial(_conv3x3_kernel, Hb=Hb, W=W),
        grid=(B, H // Hb),
        in_specs=[
            row_spec, row_spec, row_spec,
            pl.BlockSpec((3, 3, cin, cout), lambda bi, hi: (0, 0, 0, 0)),
            pl.BlockSpec((1, cout), lambda bi, hi: (0, 0)),
        ],
        out_specs=pl.BlockSpec((1, Hb * W, cout), lambda bi, hi: (bi, hi, 0)),
        out_shape=jax.ShapeDtypeStruct((B, H * W, cout), jnp.float32),
    )(*xs, w, b.reshape(1, cout))


def _moe_kernel(tid_ref, x_ref, rfeat_ref, wt_ref, temb_ref, lnw_ref, lnb_ref,
                wr_ref, w1c_ref, b1c_ref, w2v_ref, b2_ref,
                y_ref, me_ref, fe_ref):
    b = pl.program_id(0)
    i = pl.program_id(1)
    d = x_ref.shape[-1]
    x = x_ref[...]

    mu = jnp.mean(x, axis=1, keepdims=True)
    var = jnp.mean((x - mu) ** 2, axis=1, keepdims=True)
    t = (x - mu) * jax.lax.rsqrt(var + 1e-6) * lnw_ref[...] + lnb_ref[...]

    rf_all = _dot(rfeat_ref[...], wt_ref[...])
    rows_b = jax.lax.broadcasted_iota(jnp.int32, rf_all.shape, 0)
    rb = jnp.sum(jnp.where(rows_b == b, rf_all, 0.0), axis=0, keepdims=True)
    rows_t = jax.lax.broadcasted_iota(jnp.int32, temb_ref.shape, 0)
    te = jnp.sum(jnp.where(rows_t == tid_ref[0], temb_ref[...], 0.0),
                 axis=0, keepdims=True)
    logits = _dot(t, wr_ref[...]) + rb + te

    v1 = jnp.max(logits, axis=1, keepdims=True)
    ex = jnp.exp(logits - v1)
    probs = ex / jnp.sum(ex, axis=1, keepdims=True)

    lane = jax.lax.broadcasted_iota(jnp.int32, logits.shape, 1)
    i1 = jnp.min(jnp.where(logits >= v1, lane, _NE), axis=1, keepdims=True)
    masked = jnp.where(lane == i1, -1e30, logits)
    v2 = jnp.max(masked, axis=1, keepdims=True)
    i2 = jnp.min(jnp.where(masked >= v2, lane, _NE), axis=1, keepdims=True)
    g1 = 1.0 / (1.0 + jnp.exp(v2 - v1))
    g2 = 1.0 - g1
    oh1 = (lane == i1).astype(jnp.float32)
    oh2 = (lane == i2).astype(jnp.float32)
    combine = g1 * oh1 + g2 * oh2
    dispatch = oh1 + oh2

    h = jax.nn.gelu(_dot(t, w1c_ref[...]) + b1c_ref[...])
    eol = jax.lax.broadcasted_iota(jnp.int32, h.shape, 1) // (2 * d)
    crep = jnp.where(eol == i1, g1, 0.0) + jnp.where(eol == i2, g2, 0.0)
    out = _dot(h * crep, w2v_ref[...]) + _dot(combine, b2_ref[...])
    y_ref[...] = x + out

    pme = jnp.sum(probs, axis=0, keepdims=True)
    pfe = jnp.sum(dispatch, axis=0, keepdims=True)
    first = jnp.logical_and(b == 0, i == 0)

    @pl.when(first)
    def _():
        me_ref[...] = jnp.broadcast_to(pme, me_ref.shape)
        fe_ref[...] = jnp.broadcast_to(pfe, fe_ref.shape)

    @pl.when(jnp.logical_not(first))
    def _():
        me_ref[...] += jnp.broadcast_to(pme, me_ref.shape)
        fe_ref[...] += jnp.broadcast_to(pfe, fe_ref.shape)


def _moe(tokens, rfeat, tid, p, hw, tb):
    T, d = tokens.shape
    B = T // hw
    nb = hw // tb
    w1c = p['W1'].transpose(1, 0, 2).reshape(d, _NE * 2 * d)
    b1c = p['b1'].reshape(1, _NE * 2 * d)
    w2v = p['W2'].reshape(_NE * 2 * d, d)
    full = lambda arr: pl.BlockSpec(arr.shape, lambda b_, i_, n=arr.ndim: (0,) * n)
    y, me, fe = pl.pallas_call(
        _moe_kernel,
        grid=(B, nb),
        in_specs=[
            pl.BlockSpec(memory_space=pltpu.SMEM),
            pl.BlockSpec((tb, d), lambda b_, i_: (b_ * nb + i_, 0)),
            full(rfeat), full(p['Wt']), full(p['task_emb']),
            pl.BlockSpec((1, d), lambda b_, i_: (0, 0)),
            pl.BlockSpec((1, d), lambda b_, i_: (0, 0)),
            full(p['Wr']), full(w1c), full(b1c), full(w2v), full(p['b2']),
        ],
        out_specs=[
            pl.BlockSpec((tb, d), lambda b_, i_: (b_ * nb + i_, 0)),
            pl.BlockSpec((8, _NE), lambda b_, i_: (0, 0)),
            pl.BlockSpec((8, _NE), lambda b_, i_: (0, 0)),
        ],
        out_shape=[
            jax.ShapeDtypeStruct((T, d), jnp.float32),
            jax.ShapeDtypeStruct((8, _NE), jnp.float32),
            jax.ShapeDtypeStruct((8, _NE), jnp.float32),
        ],
    )(tid, tokens, rfeat, p['Wt'], p['task_emb'],
      p['ln_w'].reshape(1, d), p['ln_b'].reshape(1, d),
      p['Wr'], w1c, b1c, w2v, p['b2'])
    return y, me, fe


def _loss_kernel(*refs, counts):
    std_ref, mi_ref = refs[-2], refs[-1]
    std = jnp.zeros((1, 1), jnp.float32)
    mi = jnp.zeros((1, 1), jnp.float32)
    for k, cnt in enumerate(counts):
        me = refs[2 * k][0:1, :] * (1.0 / cnt)
        fe = refs[2 * k + 1][0:1, :] * (1.0 / cnt)
        std = std + float(_NE) * jnp.sum(me * fe, keepdims=True)
        mi = mi + jnp.sum(me * jnp.log(me + 1e-9), keepdims=True)
    std_ref[...] = std
    mi_ref[...] = mi


def _losses(mefes, counts):
    full = lambda arr: pl.BlockSpec(arr.shape, lambda: (0, 0))
    std, mi = pl.pallas_call(
        functools.partial(_loss_kernel, counts=tuple(float(c) for c in counts)),
        in_specs=[full(a) for a in mefes],
        out_specs=[pl.BlockSpec((1, 1), lambda: (0, 0))] * 2,
        out_shape=[jax.ShapeDtypeStruct((1, 1), jnp.float32)] * 2,
    )(*mefes)
    return std.reshape(()), mi.reshape(())


def _unshuffle(tok, B, H, W, C):
    t = tok.reshape(B, H // 2, 2, W // 2, 2, C)
    return t.transpose(0, 1, 3, 5, 2, 4).reshape(B, H // 2, W // 2, 4 * C)


def kernel(x, route_feature, task_id, params):
    p = params
    B = x.shape[0]
    tid = jnp.asarray(task_id, jnp.int32).reshape(1)

    t1 = _conv3x3(x.transpose(0, 2, 3, 1), p['conv0_w'], p['conv0_b'])
    x1 = t1.reshape(B, 224, 224, 16).transpose(0, 3, 1, 2)
    y1, me1, fe1 = _moe(t1.reshape(B * 50176, 16), route_feature, tid,
                        p['blk1'], 50176, 1792)

    c1 = _conv3x3(y1.reshape(B, 224, 224, 16), p['down1_w'], None)
    t2 = _unshuffle(c1, B, 224, 224, 8)
    x2 = t2.transpose(0, 3, 1, 2)
    y2, me2, fe2 = _moe(t2.reshape(B * 12544, 32), route_feature, tid,
                        p['blk2'], 12544, 1568)

    c2 = _conv3x3(y2.reshape(B, 112, 112, 32), p['down2_w'], None)
    t3 = _unshuffle(c2, B, 112, 112, 16)
    x3 = t3.transpose(0, 3, 1, 2)
    y3, me3, fe3 = _moe(t3.reshape(B * 3136, 64), route_feature, tid,
                        p['blk3'], 3136, 1568)

    c3 = _conv3x3(y3.reshape(B, 56, 56, 64), p['down3_w'], None)
    t4 = _unshuffle(c3, B, 56, 56, 32)
    x4 = t4.transpose(0, 3, 1, 2)
    y4, me4, fe4 = _moe(t4.reshape(B * 784, 128), route_feature, tid,
                        p['blk4'], 784, 784)

    c4 = _conv3x3(y4.reshape(B, 28, 28, 128), p['down4_w'], None)
    t5 = _unshuffle(c4, B, 28, 28, 64)
    x5 = t5.transpose(0, 3, 1, 2)

    counts = [B * 50176, B * 12544, B * 3136, B * 784]
    std, mi = _losses([me1, fe1, me2, fe2, me3, fe3, me4, fe4], counts)
    return x1, x2, x3, x4, x5, std, mi


# speedup vs baseline: 2.8524x; 2.0286x over previous
import functools

import jax
import jax.numpy as jnp
from jax.experimental import pallas as pl
from jax.experimental.pallas import tpu as pltpu

_NE = 8


def _dot1(a, b):
    return jax.lax.dot_general(
        a, b, (((a.ndim - 1,), (0,)), ((), ())),
        preferred_element_type=jnp.float32)


def _dot(a, b):
    ah = a.astype(jnp.bfloat16)
    al = (a - ah.astype(jnp.float32)).astype(jnp.bfloat16)
    bh = b.astype(jnp.bfloat16)
    bl = (b - bh.astype(jnp.float32)).astype(jnp.bfloat16)
    return _dot1(ah, bh) + (_dot1(ah, bl) + _dot1(al, bh))


def _conv3x3_kernel(x_ref, w_ref, b_ref, out_ref, *, Hb, W, Wp):
    cout = w_ref.shape[-1] // 3
    xb = x_ref[0]
    p = _dot(xb.reshape(Hb * Wp, xb.shape[-1]), w_ref[...])
    p = p.reshape(Hb, Wp, 3 * cout)
    acc = (p[:, 0:W, 0:cout] + p[:, 1:W + 1, cout:2 * cout]
           + p[:, 2:W + 2, 2 * cout:3 * cout])
    out_ref[0] = acc.reshape(Hb * W, cout) + b_ref[...]


def _conv3x3(x_nhwc, w_oihw, bias):
    B, H, W, cin = x_nhwc.shape
    cout = w_oihw.shape[0]
    Hb = 14
    Wp = W + 8
    xp = jnp.pad(x_nhwc, ((0, 0), (1, 1), (1, 7), (0, 0)))
    xcat = jnp.concatenate([xp[:, dy:dy + H] for dy in range(3)], axis=-1)
    w = w_oihw.transpose(2, 1, 3, 0).reshape(3 * cin, 3 * cout)
    b = bias if bias is not None else jnp.zeros((cout,), jnp.float32)
    return pl.pallas_call(
        functools.partial(_conv3x3_kernel, Hb=Hb, W=W, Wp=Wp),
        grid=(B, H // Hb),
        in_specs=[
            pl.BlockSpec((1, Hb, Wp, 3 * cin), lambda bi, hi: (bi, hi, 0, 0)),
            pl.BlockSpec((3 * cin, 3 * cout), lambda bi, hi: (0, 0)),
            pl.BlockSpec((1, cout), lambda bi, hi: (0, 0)),
        ],
        out_specs=pl.BlockSpec((1, Hb * W, cout), lambda bi, hi: (bi, hi, 0)),
        out_shape=jax.ShapeDtypeStruct((B, H * W, cout), jnp.float32),
    )(xcat, w, b.reshape(1, cout))


def _moe_kernel(tid_ref, x_ref, rfeat_ref, wt_ref, temb_ref, lnw_ref, lnb_ref,
                wr_ref, w1c_ref, b1c_ref, w2v_ref, b2_ref, rep_ref,
                y_ref, me_ref, fe_ref):
    b = pl.program_id(0)
    i = pl.program_id(1)
    d = x_ref.shape[-1]
    x = x_ref[...]

    ones = jnp.ones((d, 1), jnp.float32)
    mu = _dot(x, ones) * (1.0 / d)
    var = _dot(x * x, ones) * (1.0 / d) - mu * mu
    t = (x - mu) * jax.lax.rsqrt(var + 1e-6) * lnw_ref[...] + lnb_ref[...]

    rf_all = _dot(rfeat_ref[...], wt_ref[...])
    rows_b = jax.lax.broadcasted_iota(jnp.int32, rf_all.shape, 0)
    rb = jnp.sum(jnp.where(rows_b == b, rf_all, 0.0), axis=0, keepdims=True)
    rows_t = jax.lax.broadcasted_iota(jnp.int32, temb_ref.shape, 0)
    te = jnp.sum(jnp.where(rows_t == tid_ref[0], temb_ref[...], 0.0),
                 axis=0, keepdims=True)
    logits = _dot(t, wr_ref[...]) + rb + te

    v1 = jnp.max(logits, axis=1, keepdims=True)
    ex = jnp.exp(logits - v1)
    probs = ex / jnp.sum(ex, axis=1, keepdims=True)

    lane = jax.lax.broadcasted_iota(jnp.int32, logits.shape, 1)
    i1 = jnp.min(jnp.where(logits >= v1, lane, _NE), axis=1, keepdims=True)
    masked = jnp.where(lane == i1, -1e30, logits)
    v2 = jnp.max(masked, axis=1, keepdims=True)
    i2 = jnp.min(jnp.where(masked >= v2, lane, _NE), axis=1, keepdims=True)
    g1 = 1.0 / (1.0 + jnp.exp(v2 - v1))
    g2 = 1.0 - g1
    oh1 = (lane == i1).astype(jnp.float32)
    oh2 = (lane == i2).astype(jnp.float32)
    combine = g1 * oh1 + g2 * oh2
    dispatch = oh1 + oh2

    h = jax.nn.gelu(_dot(t, w1c_ref[...]) + b1c_ref[...])
    crep = _dot(combine, rep_ref[...])
    out = _dot(h * crep, w2v_ref[...]) + _dot(combine, b2_ref[...])
    y_ref[...] = x + out

    pme = jnp.sum(probs, axis=0, keepdims=True)
    pfe = jnp.sum(dispatch, axis=0, keepdims=True)
    first = jnp.logical_and(b == 0, i == 0)

    @pl.when(first)
    def _():
        me_ref[...] = jnp.broadcast_to(pme, me_ref.shape)
        fe_ref[...] = jnp.broadcast_to(pfe, fe_ref.shape)

    @pl.when(jnp.logical_not(first))
    def _():
        me_ref[...] += jnp.broadcast_to(pme, me_ref.shape)
        fe_ref[...] += jnp.broadcast_to(pfe, fe_ref.shape)


def _moe(tokens, rfeat, tid, p, hw, tb):
    T, d = tokens.shape
    B = T // hw
    nb = hw // tb
    w1c = p['W1'].transpose(1, 0, 2).reshape(d, _NE * 2 * d)
    b1c = p['b1'].reshape(1, _NE * 2 * d)
    w2v = p['W2'].reshape(_NE * 2 * d, d)
    rep = jnp.repeat(jnp.eye(_NE, dtype=jnp.float32), 2 * d, axis=1)
    full = lambda arr: pl.BlockSpec(arr.shape, lambda b_, i_, n=arr.ndim: (0,) * n)
    y, me, fe = pl.pallas_call(
        _moe_kernel,
        grid=(B, nb),
        in_specs=[
            pl.BlockSpec(memory_space=pltpu.SMEM),
            pl.BlockSpec((tb, d), lambda b_, i_: (b_ * nb + i_, 0)),
            full(rfeat), full(p['Wt']), full(p['task_emb']),
            pl.BlockSpec((1, d), lambda b_, i_: (0, 0)),
            pl.BlockSpec((1, d), lambda b_, i_: (0, 0)),
            full(p['Wr']), full(w1c), full(b1c), full(w2v), full(p['b2']),
            full(rep),
        ],
        out_specs=[
            pl.BlockSpec((tb, d), lambda b_, i_: (b_ * nb + i_, 0)),
            pl.BlockSpec((8, _NE), lambda b_, i_: (0, 0)),
            pl.BlockSpec((8, _NE), lambda b_, i_: (0, 0)),
        ],
        out_shape=[
            jax.ShapeDtypeStruct((T, d), jnp.float32),
            jax.ShapeDtypeStruct((8, _NE), jnp.float32),
            jax.ShapeDtypeStruct((8, _NE), jnp.float32),
        ],
    )(tid, tokens, rfeat, p['Wt'], p['task_emb'],
      p['ln_w'].reshape(1, d), p['ln_b'].reshape(1, d),
      p['Wr'], w1c, b1c, w2v, p['b2'], rep)
    return y, me, fe


def _loss_kernel(*refs, counts):
    std_ref, mi_ref = refs[-2], refs[-1]
    std = jnp.zeros((1, 1), jnp.float32)
    mi = jnp.zeros((1, 1), jnp.float32)
    for k, cnt in enumerate(counts):
        me = refs[2 * k][0:1, :] * (1.0 / cnt)
        fe = refs[2 * k + 1][0:1, :] * (1.0 / cnt)
        std = std + float(_NE) * jnp.sum(me * fe, keepdims=True)
        mi = mi + jnp.sum(me * jnp.log(me + 1e-9), keepdims=True)
    std_ref[...] = std
    mi_ref[...] = mi


def _losses(mefes, counts):
    full = lambda arr: pl.BlockSpec(arr.shape, lambda: (0, 0))
    std, mi = pl.pallas_call(
        functools.partial(_loss_kernel, counts=tuple(float(c) for c in counts)),
        in_specs=[full(a) for a in mefes],
        out_specs=[pl.BlockSpec((1, 1), lambda: (0, 0))] * 2,
        out_shape=[jax.ShapeDtypeStruct((1, 1), jnp.float32)] * 2,
    )(*mefes)
    return std.reshape(()), mi.reshape(())


def _unshuffle(tok, B, H, W, C):
    t = tok.reshape(B, H // 2, 2, W // 2, 2, C)
    return t.transpose(0, 1, 3, 5, 2, 4).reshape(B, H // 2, W // 2, 4 * C)


def kernel(x, route_feature, task_id, params):
    p = params
    B = x.shape[0]
    tid = jnp.asarray(task_id, jnp.int32).reshape(1)

    t1 = _conv3x3(x.transpose(0, 2, 3, 1), p['conv0_w'], p['conv0_b'])
    x1 = t1.reshape(B, 224, 224, 16).transpose(0, 3, 1, 2)
    y1, me1, fe1 = _moe(t1.reshape(B * 50176, 16), route_feature, tid,
                        p['blk1'], 50176, 1792)

    c1 = _conv3x3(y1.reshape(B, 224, 224, 16), p['down1_w'], None)
    t2 = _unshuffle(c1, B, 224, 224, 8)
    x2 = t2.transpose(0, 3, 1, 2)
    y2, me2, fe2 = _moe(t2.reshape(B * 12544, 32), route_feature, tid,
                        p['blk2'], 12544, 1568)

    c2 = _conv3x3(y2.reshape(B, 112, 112, 32), p['down2_w'], None)
    t3 = _unshuffle(c2, B, 112, 112, 16)
    x3 = t3.transpose(0, 3, 1, 2)
    y3, me3, fe3 = _moe(t3.reshape(B * 3136, 64), route_feature, tid,
                        p['blk3'], 3136, 1568)

    c3 = _conv3x3(y3.reshape(B, 56, 56, 64), p['down3_w'], None)
    t4 = _unshuffle(c3, B, 56, 56, 32)
    x4 = t4.transpose(0, 3, 1, 2)
    y4, me4, fe4 = _moe(t4.reshape(B * 784, 128), route_feature, tid,
                        p['blk4'], 784, 784)

    c4 = _conv3x3(y4.reshape(B, 28, 28, 128), p['down4_w'], None)
    t5 = _unshuffle(c4, B, 28, 28, 64)
    x5 = t5.transpose(0, 3, 1, 2)

    counts = [B * 50176, B * 12544, B * 3136, B * 784]
    std, mi = _losses([me1, fe1, me2, fe2, me3, fe3, me4, fe4], counts)
    return x1, x2, x3, x4, x5, std, mi


# speedup vs baseline: 3.1551x; 1.1061x over previous
import functools

import jax
import jax.numpy as jnp
from jax.experimental import pallas as pl
from jax.experimental.pallas import tpu as pltpu

_NE = 8


def _dot1(a, b):
    return jax.lax.dot_general(
        a, b, (((a.ndim - 1,), (0,)), ((), ())),
        preferred_element_type=jnp.float32)


def _dot(a, b):
    ah = a.astype(jnp.bfloat16)
    al = (a - ah.astype(jnp.float32)).astype(jnp.bfloat16)
    bh = b.astype(jnp.bfloat16)
    bl = (b - bh.astype(jnp.float32)).astype(jnp.bfloat16)
    return _dot1(ah, bh) + (_dot1(ah, bl) + _dot1(al, bh))


def _conv3x3_kernel(x_ref, w_ref, b_ref, out_ref, *, Hb, W, Wp):
    cout = w_ref.shape[-1] // 3
    xb = x_ref[0]
    p = _dot(xb.reshape(Hb * Wp, xb.shape[-1]), w_ref[...])
    p = p.reshape(Hb, Wp, 3 * cout)
    acc = (p[:, 0:W, 0:cout] + p[:, 1:W + 1, cout:2 * cout]
           + p[:, 2:W + 2, 2 * cout:3 * cout])
    out_ref[0] = acc.reshape(Hb * W, cout) + b_ref[...]


def _conv3x3(x_nhwc, w_oihw, bias):
    B, H, W, cin = x_nhwc.shape
    cout = w_oihw.shape[0]
    Hb = 28
    Wp = W + 8
    xp = jnp.pad(x_nhwc, ((0, 0), (1, 1), (1, 7), (0, 0)))
    xcat = jnp.concatenate([xp[:, dy:dy + H] for dy in range(3)], axis=-1)
    w = w_oihw.transpose(2, 1, 3, 0).reshape(3 * cin, 3 * cout)
    b = bias if bias is not None else jnp.zeros((cout,), jnp.float32)
    return pl.pallas_call(
        functools.partial(_conv3x3_kernel, Hb=Hb, W=W, Wp=Wp),
        grid=(B, H // Hb),
        in_specs=[
            pl.BlockSpec((1, Hb, Wp, 3 * cin), lambda bi, hi: (bi, hi, 0, 0)),
            pl.BlockSpec((3 * cin, 3 * cout), lambda bi, hi: (0, 0)),
            pl.BlockSpec((1, cout), lambda bi, hi: (0, 0)),
        ],
        out_specs=pl.BlockSpec((1, Hb * W, cout), lambda bi, hi: (bi, hi, 0)),
        out_shape=jax.ShapeDtypeStruct((B, H * W, cout), jnp.float32),
    )(xcat, w, b.reshape(1, cout))


def _moe_kernel(tid_ref, x_ref, rfeat_ref, wt_ref, temb_ref, lnw_ref, lnb_ref,
                wr_ref, w1c_ref, b1c_ref, w2v_ref, b2_ref,
                y_ref, me_ref, fe_ref):
    b = pl.program_id(0)
    i = pl.program_id(1)
    d = x_ref.shape[-1]
    x = x_ref[...]

    mu = jnp.mean(x, axis=1, keepdims=True)
    var = jnp.mean((x - mu) ** 2, axis=1, keepdims=True)
    t = (x - mu) * jax.lax.rsqrt(var + 1e-6) * lnw_ref[...] + lnb_ref[...]

    rf_all = _dot(rfeat_ref[...], wt_ref[...])
    rows_b = jax.lax.broadcasted_iota(jnp.int32, rf_all.shape, 0)
    rb = jnp.sum(jnp.where(rows_b == b, rf_all, 0.0), axis=0, keepdims=True)
    rows_t = jax.lax.broadcasted_iota(jnp.int32, temb_ref.shape, 0)
    te = jnp.sum(jnp.where(rows_t == tid_ref[0], temb_ref[...], 0.0),
                 axis=0, keepdims=True)
    logits = _dot(t, wr_ref[...]) + rb + te

    v1 = jnp.max(logits, axis=1, keepdims=True)
    ex = jnp.exp(logits - v1)
    probs = ex / jnp.sum(ex, axis=1, keepdims=True)

    lane = jax.lax.broadcasted_iota(jnp.int32, logits.shape, 1)
    i1 = jnp.min(jnp.where(logits >= v1, lane, _NE), axis=1, keepdims=True)
    masked = jnp.where(lane == i1, -1e30, logits)
    v2 = jnp.max(masked, axis=1, keepdims=True)
    i2 = jnp.min(jnp.where(masked >= v2, lane, _NE), axis=1, keepdims=True)
    g1 = 1.0 / (1.0 + jnp.exp(v2 - v1))
    g2 = 1.0 - g1
    oh1 = (lane == i1).astype(jnp.float32)
    oh2 = (lane == i2).astype(jnp.float32)
    combine = g1 * oh1 + g2 * oh2
    dispatch = oh1 + oh2

    h = jax.nn.gelu(_dot(t, w1c_ref[...]) + b1c_ref[...])
    eol = jax.lax.broadcasted_iota(jnp.int32, h.shape, 1) // (2 * d)
    crep = jnp.where(eol == i1, g1, 0.0) + jnp.where(eol == i2, g2, 0.0)
    out = _dot(h * crep, w2v_ref[...]) + _dot(combine, b2_ref[...])
    y_ref[...] = x + out

    pme = jnp.sum(probs, axis=0, keepdims=True)
    pfe = jnp.sum(dispatch, axis=0, keepdims=True)
    first = jnp.logical_and(b == 0, i == 0)

    @pl.when(first)
    def _():
        me_ref[...] = jnp.broadcast_to(pme, me_ref.shape)
        fe_ref[...] = jnp.broadcast_to(pfe, fe_ref.shape)

    @pl.when(jnp.logical_not(first))
    def _():
        me_ref[...] += jnp.broadcast_to(pme, me_ref.shape)
        fe_ref[...] += jnp.broadcast_to(pfe, fe_ref.shape)


def _moe(tokens, rfeat, tid, p, hw, tb):
    T, d = tokens.shape
    B = T // hw
    nb = hw // tb
    w1c = p['W1'].transpose(1, 0, 2).reshape(d, _NE * 2 * d)
    b1c = p['b1'].reshape(1, _NE * 2 * d)
    w2v = p['W2'].reshape(_NE * 2 * d, d)
    full = lambda arr: pl.BlockSpec(arr.shape, lambda b_, i_, n=arr.ndim: (0,) * n)
    y, me, fe = pl.pallas_call(
        _moe_kernel,
        grid=(B, nb),
        in_specs=[
            pl.BlockSpec(memory_space=pltpu.SMEM),
            pl.BlockSpec((tb, d), lambda b_, i_: (b_ * nb + i_, 0)),
            full(rfeat), full(p['Wt']), full(p['task_emb']),
            pl.BlockSpec((1, d), lambda b_, i_: (0, 0)),
            pl.BlockSpec((1, d), lambda b_, i_: (0, 0)),
            full(p['Wr']), full(w1c), full(b1c), full(w2v), full(p['b2']),
        ],
        out_specs=[
            pl.BlockSpec((tb, d), lambda b_, i_: (b_ * nb + i_, 0)),
            pl.BlockSpec((8, _NE), lambda b_, i_: (0, 0)),
            pl.BlockSpec((8, _NE), lambda b_, i_: (0, 0)),
        ],
        out_shape=[
            jax.ShapeDtypeStruct((T, d), jnp.float32),
            jax.ShapeDtypeStruct((8, _NE), jnp.float32),
            jax.ShapeDtypeStruct((8, _NE), jnp.float32),
        ],
    )(tid, tokens, rfeat, p['Wt'], p['task_emb'],
      p['ln_w'].reshape(1, d), p['ln_b'].reshape(1, d),
      p['Wr'], w1c, b1c, w2v, p['b2'])
    return y, me, fe


def _loss_kernel(*refs, counts):
    std_ref, mi_ref = refs[-2], refs[-1]
    std = jnp.zeros((1, 1), jnp.float32)
    mi = jnp.zeros((1, 1), jnp.float32)
    for k, cnt in enumerate(counts):
        me = refs[2 * k][0:1, :] * (1.0 / cnt)
        fe = refs[2 * k + 1][0:1, :] * (1.0 / cnt)
        std = std + float(_NE) * jnp.sum(me * fe, keepdims=True)
        mi = mi + jnp.sum(me * jnp.log(me + 1e-9), keepdims=True)
    std_ref[...] = std
    mi_ref[...] = mi


def _losses(mefes, counts):
    full = lambda arr: pl.BlockSpec(arr.shape, lambda: (0, 0))
    std, mi = pl.pallas_call(
        functools.partial(_loss_kernel, counts=tuple(float(c) for c in counts)),
        in_specs=[full(a) for a in mefes],
        out_specs=[pl.BlockSpec((1, 1), lambda: (0, 0))] * 2,
        out_shape=[jax.ShapeDtypeStruct((1, 1), jnp.float32)] * 2,
    )(*mefes)
    return std.reshape(()), mi.reshape(())


def _unshuffle(tok, B, H, W, C):
    t = tok.reshape(B, H // 2, 2, W // 2, 2, C)
    return t.transpose(0, 1, 3, 5, 2, 4).reshape(B, H // 2, W // 2, 4 * C)


def kernel(x, route_feature, task_id, params):
    p = params
    B = x.shape[0]
    tid = jnp.asarray(task_id, jnp.int32).reshape(1)

    t1 = _conv3x3(x.transpose(0, 2, 3, 1), p['conv0_w'], p['conv0_b'])
    x1 = t1.reshape(B, 224, 224, 16).transpose(0, 3, 1, 2)
    y1, me1, fe1 = _moe(t1.reshape(B * 50176, 16), route_feature, tid,
                        p['blk1'], 50176, 3584)

    c1 = _conv3x3(y1.reshape(B, 224, 224, 16), p['down1_w'], None)
    t2 = _unshuffle(c1, B, 224, 224, 8)
    x2 = t2.transpose(0, 3, 1, 2)
    y2, me2, fe2 = _moe(t2.reshape(B * 12544, 32), route_feature, tid,
                        p['blk2'], 12544, 3136)

    c2 = _conv3x3(y2.reshape(B, 112, 112, 32), p['down2_w'], None)
    t3 = _unshuffle(c2, B, 112, 112, 16)
    x3 = t3.transpose(0, 3, 1, 2)
    y3, me3, fe3 = _moe(t3.reshape(B * 3136, 64), route_feature, tid,
                        p['blk3'], 3136, 1568)

    c3 = _conv3x3(y3.reshape(B, 56, 56, 64), p['down3_w'], None)
    t4 = _unshuffle(c3, B, 56, 56, 32)
    x4 = t4.transpose(0, 3, 1, 2)
    y4, me4, fe4 = _moe(t4.reshape(B * 784, 128), route_feature, tid,
                        p['blk4'], 784, 784)

    c4 = _conv3x3(y4.reshape(B, 28, 28, 128), p['down4_w'], None)
    t5 = _unshuffle(c4, B, 28, 28, 64)
    x5 = t5.transpose(0, 3, 1, 2)

    counts = [B * 50176, B * 12544, B * 3136, B * 784]
    std, mi = _losses([me1, fe1, me2, fe2, me3, fe3, me4, fe4], counts)
    return x1, x2, x3, x4, x5, std, mi
